# baseline (speedup 1.0000x reference)
import functools

import jax
import jax.numpy as jnp
import numpy as np
from jax import lax
from jax.experimental import pallas as pl
from jax.experimental.pallas import tpu as pltpu
from jax.experimental.pallas import tpu_sc as plsc

_N_NODES = 1000000
_N_DIM = 64
_ROW = _N_DIM + 1
_RADIUS = 10.0
_SIGMA = 1.0
_BATCH = 16384

_NC = 2
_NS = 16
_NW = _NC * _NS
_PAIRS_PER_W = _BATCH // _NW
_ROWS_PER_W = 2 * _PAIRS_PER_W


def _trapz_np(y, x):
    dx = x[1:] - x[:-1]
    return float(np.sum((y[1:] + y[:-1]) * 0.5 * dx))


def _lik_const():
    theta = np.linspace(1e-6, np.pi - 1e-6, 20001)
    s = 0.0
    for j in range(1, _N_DIM - 1):
        s += np.log(_trapz_np(np.sin(theta) ** (_N_DIM - 1 - j), theta))
    r = np.linspace(1e-8, _RADIUS, 20001)
    logf = (_N_DIM - 1) * np.log(np.sinh(_SIGMA * r) + 1e-300)
    m = logf.max()
    log_c_d = float(m + np.log(_trapz_np(np.exp(logf - m), r)))
    return float(s + log_c_d + np.log(2.0 * np.pi))


_LIK_CONST = _lik_const()
_LOG2 = float(np.log(2.0))


_HALF = _ROWS_PER_W // 2
_HGROUPS = _HALF // 32


def _sc_reduce_body(pairs_hbm, table_hbm, p_hbm, su_hbm, sv_hbm,
                    idx_v, rows_v, p_loc, su_loc, sv_loc, sem):
    wid = lax.axis_index("s") * _NC + lax.axis_index("c")

    pltpu.sync_copy(pairs_hbm.at[wid], idx_v)

    lanes = lax.iota(jnp.int32, 16)

    for h in range(2):
        def fire(g, carry):
            idx16 = idx_v[pl.ds(h * _HALF + g * 16, 16)]
            for j in range(16):
                r = jnp.max(jnp.where(lanes == j, idx16, 0))
                pltpu.async_copy(
                    table_hbm.at[pl.ds(r, 1)],
                    rows_v.at[pl.ds(g * 16 + j, 1)],
                    sem,
                )
            return carry

        lax.fori_loop(0, _HALF // 16, fire, 0)
        pltpu.make_async_copy(
            table_hbm.at[pl.ds(0, _HALF)],
            rows_v.at[pl.ds(0, _HALF)],
            sem,
        ).wait()

        def group(g, carry):
            ru = g * 32 + lanes * 2
            rv = ru + 1
            col0 = jnp.zeros((16,), jnp.int32)
            u0 = plsc.load_gather(rows_v, [ru, col0])
            v0 = plsc.load_gather(rows_v, [rv, col0])
            duv = jnp.zeros((16,), jnp.float32)
            su = jnp.zeros((16,), jnp.float32)
            sv = jnp.zeros((16,), jnp.float32)
            for d in range(1, _ROW):
                cd = jnp.full((16,), d, jnp.int32)
                ud = plsc.load_gather(rows_v, [ru, cd])
                vd = plsc.load_gather(rows_v, [rv, cd])
                duv = duv + ud * vd
                su = su + ud * ud
                sv = sv + vd * vd
            base = h * (_HALF // 2) + g * 16
            p_loc[pl.ds(base, 16)] = u0 * v0 - duv
            su_loc[pl.ds(base, 16)] = su
            sv_loc[pl.ds(base, 16)] = sv
            return carry

        lax.fori_loop(0, _HGROUPS, group, 0)

    base = wid * _PAIRS_PER_W
    pltpu.sync_copy(p_loc, p_hbm.at[pl.ds(base, _PAIRS_PER_W)])
    pltpu.sync_copy(su_loc, su_hbm.at[pl.ds(base, _PAIRS_PER_W)])
    pltpu.sync_copy(sv_loc, sv_hbm.at[pl.ds(base, _PAIRS_PER_W)])


@functools.cache
def _sc_reduce():
    return pl.kernel(
        _sc_reduce_body,
        out_type=(
            jax.ShapeDtypeStruct((_BATCH,), jnp.float32),
            jax.ShapeDtypeStruct((_BATCH,), jnp.float32),
            jax.ShapeDtypeStruct((_BATCH,), jnp.float32),
        ),
        mesh=plsc.VectorSubcoreMesh(
            core_axis_name="c", subcore_axis_name="s",
            num_cores=_NC, num_subcores=_NS),
        compiler_params=pltpu.CompilerParams(
            needs_layout_passes=False, use_tc_tiling_on_sc=False),
        scratch_types=[
            pltpu.VMEM((_ROWS_PER_W,), jnp.int32),
            pltpu.VMEM((_HALF, _ROW), jnp.float32),
            pltpu.VMEM((_PAIRS_PER_W,), jnp.float32),
            pltpu.VMEM((_PAIRS_PER_W,), jnp.float32),
            pltpu.VMEM((_PAIRS_PER_W,), jnp.float32),
            pltpu.SemaphoreType.DMA,
        ],
    )


def _softplus(z):
    return jnp.maximum(z, 0.0) + jnp.log1p(jnp.exp(-jnp.abs(z)))


def _latent_lik_from_s(s):
    r = jnp.log(jnp.sqrt(1.0 + s) + jnp.sqrt(s))
    r = jnp.where(r <= 1e-6, 1e-6, r)
    e2r = jnp.exp(-2.0 * r)
    lik = -(_N_DIM - 1) * (jnp.log(1.0 - e2r + 1e-5) + _SIGMA * r - _LOG2)
    lik = lik + _LIK_CONST
    lik = lik + (_N_DIM - 1) * (jnp.log(1.0 - e2r + 1e-5) + r - _LOG2)
    lik = lik + jnp.log(1.0 + e2r + 1e-5) + r - _LOG2
    return lik


def _tc_body(beta_ref, labels_ref, p_ref, su_ref, sv_ref, out_ref):
    beta = beta_ref[0]
    p = jnp.maximum(p_ref[...], 1.0 + 1e-7)
    dist = jnp.log(p + jnp.sqrt((p - 1.0) * (p + 1.0)))
    z = beta * (dist - _RADIUS)
    lf = labels_ref[...].astype(jnp.float32)
    loss = jnp.where(lf == 1.0, _softplus(z), _softplus(-z))
    lik = _latent_lik_from_s(su_ref[...]) + _latent_lik_from_s(sv_ref[...])
    out_ref[...] = loss + lik / (_N_NODES - 1)


def kernel(pairs, labels, table, beta):
    pairs_per_w = pairs.reshape(_NW, _ROWS_PER_W)
    p, su, sv = _sc_reduce()(pairs_per_w, table)

    shape2d = (_BATCH // 128, 128)
    loss2d = pl.pallas_call(
        _tc_body,
        out_shape=jax.ShapeDtypeStruct(shape2d, jnp.float32),
        in_specs=[
            pl.BlockSpec(memory_space=pltpu.SMEM),
            pl.BlockSpec(),
            pl.BlockSpec(),
            pl.BlockSpec(),
            pl.BlockSpec(),
        ],
    )(beta.reshape(1), labels.reshape(shape2d), p.reshape(shape2d),
      su.reshape(shape2d), sv.reshape(shape2d))
    return loss2d.reshape(_BATCH)


# speedup vs baseline: 4.5771x; 4.5771x over previous
import functools

import jax
import jax.numpy as jnp
import numpy as np
from jax import lax
from jax.experimental import pallas as pl
from jax.experimental.pallas import tpu as pltpu
from jax.experimental.pallas import tpu_sc as plsc

_N_NODES = 1000000
_N_DIM = 64
_ROW = _N_DIM + 1
_RADIUS = 10.0
_SIGMA = 1.0
_BATCH = 16384

_NC = 2
_NS = 16
_NW = _NC * _NS
_PAIRS_PER_W = _BATCH // _NW
_ROWS_PER_W = 2 * _PAIRS_PER_W


def _trapz_np(y, x):
    dx = x[1:] - x[:-1]
    return float(np.sum((y[1:] + y[:-1]) * 0.5 * dx))


def _lik_const():
    theta = np.linspace(1e-6, np.pi - 1e-6, 20001)
    s = 0.0
    for j in range(1, _N_DIM - 1):
        s += np.log(_trapz_np(np.sin(theta) ** (_N_DIM - 1 - j), theta))
    r = np.linspace(1e-8, _RADIUS, 20001)
    logf = (_N_DIM - 1) * np.log(np.sinh(_SIGMA * r) + 1e-300)
    m = logf.max()
    log_c_d = float(m + np.log(_trapz_np(np.exp(logf - m), r)))
    return float(s + log_c_d + np.log(2.0 * np.pi))


_LIK_CONST = _lik_const()
_LOG2 = float(np.log(2.0))


_HALF = _ROWS_PER_W // 2
_HGROUPS = _HALF // 32


def _sc_reduce_body(pairs_hbm, table_hbm, p_hbm, su_hbm, sv_hbm,
                    idx_v, rows_v, p_loc, su_loc, sv_loc, sem):
    wid = lax.axis_index("s") * _NC + lax.axis_index("c")

    pltpu.sync_copy(pairs_hbm.at[pl.ds(wid * _ROWS_PER_W, _ROWS_PER_W)], idx_v)

    lanes = lax.iota(jnp.int32, 16)

    for h in range(2):
        def fire(g, carry):
            idx16 = idx_v[pl.ds(h * _HALF + g * 16, 16)]
            for j in range(16):
                r = jnp.max(jnp.where(lanes == j, idx16, 0))
                pltpu.async_copy(
                    table_hbm.at[pl.ds(r, 1)],
                    rows_v.at[pl.ds(g * 16 + j, 1)],
                    sem,
                )
            return carry

        lax.fori_loop(0, _HALF // 16, fire, 0)
        pltpu.make_async_copy(
            table_hbm.at[pl.ds(0, _HALF)],
            rows_v.at[pl.ds(0, _HALF)],
            sem,
        ).wait()

        def group(g, carry):
            ru = g * 32 + lanes * 2
            rv = ru + 1
            col0 = jnp.zeros((16,), jnp.int32)
            u0 = plsc.load_gather(rows_v, [ru, col0])
            v0 = plsc.load_gather(rows_v, [rv, col0])
            duv = jnp.zeros((16,), jnp.float32)
            su = jnp.zeros((16,), jnp.float32)
            sv = jnp.zeros((16,), jnp.float32)
            for d in range(1, _ROW):
                cd = jnp.full((16,), d, jnp.int32)
                ud = plsc.load_gather(rows_v, [ru, cd])
                vd = plsc.load_gather(rows_v, [rv, cd])
                duv = duv + ud * vd
                su = su + ud * ud
                sv = sv + vd * vd
            base = h * (_HALF // 2) + g * 16
            p_loc[pl.ds(base, 16)] = u0 * v0 - duv
            su_loc[pl.ds(base, 16)] = su
            sv_loc[pl.ds(base, 16)] = sv
            return carry

        lax.fori_loop(0, _HGROUPS, group, 0)

    base = wid * _PAIRS_PER_W
    pltpu.sync_copy(p_loc, p_hbm.at[pl.ds(base, _PAIRS_PER_W)])
    pltpu.sync_copy(su_loc, su_hbm.at[pl.ds(base, _PAIRS_PER_W)])
    pltpu.sync_copy(sv_loc, sv_hbm.at[pl.ds(base, _PAIRS_PER_W)])


@functools.cache
def _sc_reduce():
    return pl.kernel(
        _sc_reduce_body,
        out_type=(
            jax.ShapeDtypeStruct((_BATCH,), jnp.float32),
            jax.ShapeDtypeStruct((_BATCH,), jnp.float32),
            jax.ShapeDtypeStruct((_BATCH,), jnp.float32),
        ),
        mesh=plsc.VectorSubcoreMesh(
            core_axis_name="c", subcore_axis_name="s",
            num_cores=_NC, num_subcores=_NS),
        compiler_params=pltpu.CompilerParams(needs_layout_passes=False),
        scratch_types=[
            pltpu.VMEM((_ROWS_PER_W,), jnp.int32),
            pltpu.VMEM((_HALF, _ROW), jnp.float32),
            pltpu.VMEM((_PAIRS_PER_W,), jnp.float32),
            pltpu.VMEM((_PAIRS_PER_W,), jnp.float32),
            pltpu.VMEM((_PAIRS_PER_W,), jnp.float32),
            pltpu.SemaphoreType.DMA,
        ],
    )


def _softplus(z):
    return jnp.maximum(z, 0.0) + jnp.log1p(jnp.exp(-jnp.abs(z)))


def _latent_lik_from_s(s):
    r = jnp.log(jnp.sqrt(1.0 + s) + jnp.sqrt(s))
    r = jnp.where(r <= 1e-6, 1e-6, r)
    e2r = jnp.exp(-2.0 * r)
    lik = -(_N_DIM - 1) * (jnp.log(1.0 - e2r + 1e-5) + _SIGMA * r - _LOG2)
    lik = lik + _LIK_CONST
    lik = lik + (_N_DIM - 1) * (jnp.log(1.0 - e2r + 1e-5) + r - _LOG2)
    lik = lik + jnp.log(1.0 + e2r + 1e-5) + r - _LOG2
    return lik


def _tc_body(beta_ref, labels_ref, p_ref, su_ref, sv_ref, out_ref):
    beta = beta_ref[0]
    p = jnp.maximum(p_ref[...], 1.0 + 1e-7)
    dist = jnp.log(p + jnp.sqrt((p - 1.0) * (p + 1.0)))
    z = beta * (dist - _RADIUS)
    lf = labels_ref[...].astype(jnp.float32)
    loss = jnp.where(lf == 1.0, _softplus(z), _softplus(-z))
    lik = _latent_lik_from_s(su_ref[...]) + _latent_lik_from_s(sv_ref[...])
    out_ref[...] = loss + lik / (_N_NODES - 1)


def kernel(pairs, labels, table, beta):
    p, su, sv = _sc_reduce()(pairs.reshape(2 * _BATCH), table)

    shape2d = (_BATCH // 128, 128)
    loss2d = pl.pallas_call(
        _tc_body,
        out_shape=jax.ShapeDtypeStruct(shape2d, jnp.float32),
        in_specs=[
            pl.BlockSpec(memory_space=pltpu.SMEM),
            pl.BlockSpec(),
            pl.BlockSpec(),
            pl.BlockSpec(),
            pl.BlockSpec(),
        ],
    )(beta.reshape(1), labels.reshape(shape2d), p.reshape(shape2d),
      su.reshape(shape2d), sv.reshape(shape2d))
    return loss2d.reshape(_BATCH)
